# baseline (speedup 1.0000x reference)
import functools

import jax
import jax.numpy as jnp
from jax import lax
from jax.experimental import pallas as pl
from jax.experimental.pallas import tpu as pltpu
from jax.experimental.pallas import tpu_sc as plsc

NUM_CORES = 2
NBUF = 4
CHUNK = 512


@functools.partial(jax.jit, static_argnames=("n_per_c", "n_chunks", "dim"))
def _gather_call(idx_flat, table, *, n_per_c, n_chunks, dim):
    n_total = idx_flat.shape[0]
    mesh = plsc.ScalarSubcoreMesh(axis_name="c", num_cores=NUM_CORES)

    @functools.partial(
        pl.kernel,
        out_type=jax.ShapeDtypeStruct((n_total, dim), jnp.float32),
        mesh=mesh,
        scratch_types=[
            pltpu.SMEM((CHUNK,), jnp.int32),
            pltpu.VMEM_SHARED((NBUF, CHUNK, dim), jnp.float32),
            [pltpu.SemaphoreType.DMA] * NBUF,
            [pltpu.SemaphoreType.DMA] * NBUF,
            pltpu.SemaphoreType.DMA,
        ],
    )
    def gather_kernel(
        idx_hbm, table_hbm, out_hbm, idx_s, rows_v, gsems, wsems, isem
    ):
        cid = lax.axis_index("c")
        base = cid * n_per_c
        bufs = tuple(rows_v.at[b] for b in range(NBUF))

        def start_gather(c, b):
            pltpu.async_copy(
                idx_hbm.at[pl.ds(base + c * CHUNK, CHUNK)], idx_s, isem
            ).wait()

            def issue(r, carry):
                v = idx_s[r]
                pltpu.async_copy(
                    table_hbm.at[pl.ds(v, 1)],
                    bufs[b].at[pl.ds(r, 1)],
                    gsems[b],
                )
                return carry

            lax.fori_loop(0, CHUNK, issue, 0)

        def wait_gather(b):
            pltpu.make_async_copy(
                table_hbm.at[pl.ds(0, CHUNK)], bufs[b], gsems[b]
            ).wait()

        def start_write(c, b):
            pltpu.async_copy(
                bufs[b], out_hbm.at[pl.ds(base + c * CHUNK, CHUNK)], wsems[b]
            )

        def wait_write(b):
            pltpu.make_async_copy(
                bufs[b], out_hbm.at[pl.ds(0, CHUNK)], wsems[b]
            ).wait()

        start_gather(0, 0)
        start_gather(1, 1)

        def body(i, carry):
            c0 = i * NBUF
            for b in range(NBUF):
                c = c0 + b
                nb = (b + 2) % NBUF

                @pl.when(c >= 2)
                def _():
                    wait_write(nb)

                @pl.when(c + 2 < n_chunks)
                def _():
                    start_gather(c + 2, nb)

                wait_gather(b)
                start_write(c, b)
            return carry

        lax.fori_loop(0, n_chunks // NBUF, body, 0)
        wait_write((n_chunks - 2) % NBUF)
        wait_write((n_chunks - 1) % NBUF)

    return gather_kernel(idx_flat, table)


def kernel(x, table):
    b, l = x.shape
    _, dim = table.shape
    n_total = b * l
    n_per_c = n_total // NUM_CORES
    n_chunks = n_per_c // CHUNK
    idx_flat = x.reshape(n_total)
    out = _gather_call(idx_flat, table, n_per_c=n_per_c, n_chunks=n_chunks, dim=dim)
    return out.reshape(b, l, dim)


# speedup vs baseline: 1.0779x; 1.0779x over previous
import functools

import jax
import jax.numpy as jnp
from jax import lax
from jax.experimental import pallas as pl
from jax.experimental.pallas import tpu as pltpu
from jax.experimental.pallas import tpu_sc as plsc

NUM_CORES = 2
NBUF = 4
CHUNK = 512


@functools.partial(jax.jit, static_argnames=("n_per_c", "n_chunks", "dim"))
def _gather_call(idx_flat, table, *, n_per_c, n_chunks, dim):
    n_total = idx_flat.shape[0]
    mesh = plsc.ScalarSubcoreMesh(axis_name="c", num_cores=NUM_CORES)

    @functools.partial(
        pl.kernel,
        out_type=jax.ShapeDtypeStruct((n_total, dim), jnp.float32),
        mesh=mesh,
        scratch_types=[
            [pltpu.SMEM((CHUNK,), jnp.int32)] * 2,
            pltpu.VMEM_SHARED((NBUF, CHUNK, dim), jnp.float32),
            [pltpu.SemaphoreType.DMA] * NBUF,
            [pltpu.SemaphoreType.DMA] * NBUF,
            [pltpu.SemaphoreType.DMA] * 2,
        ],
    )
    def gather_kernel(
        idx_hbm, table_hbm, out_hbm, idx_s, rows_v, gsems, wsems, isems
    ):
        cid = lax.axis_index("c")
        base = cid * n_per_c
        bufs = tuple(rows_v.at[b] for b in range(NBUF))

        def prefetch_idx(c, s):
            pltpu.async_copy(
                idx_hbm.at[pl.ds(base + c * CHUNK, CHUNK)], idx_s[s], isems[s]
            )

        def wait_idx(s):
            pltpu.make_async_copy(
                idx_hbm.at[pl.ds(0, CHUNK)], idx_s[s], isems[s]
            ).wait()

        def start_gather(c, b):
            s = b % 2
            wait_idx(s)

            @pl.when(c + 1 < n_chunks)
            def _():
                prefetch_idx(c + 1, (b + 1) % 2)

            def issue(r8, carry):
                r = r8 * 8
                for u in range(8):
                    v = idx_s[s][r + u]
                    pltpu.async_copy(
                        table_hbm.at[pl.ds(v, 1)],
                        bufs[b].at[pl.ds(r + u, 1)],
                        gsems[b],
                    )
                return carry

            lax.fori_loop(0, CHUNK // 8, issue, 0)

        def wait_gather(b):
            pltpu.make_async_copy(
                table_hbm.at[pl.ds(0, CHUNK)], bufs[b], gsems[b]
            ).wait()

        def start_write(c, b):
            pltpu.async_copy(
                bufs[b], out_hbm.at[pl.ds(base + c * CHUNK, CHUNK)], wsems[b]
            )

        def wait_write(b):
            pltpu.make_async_copy(
                bufs[b], out_hbm.at[pl.ds(0, CHUNK)], wsems[b]
            ).wait()

        prefetch_idx(0, 0)
        start_gather(0, 0)
        start_gather(1, 1)

        def body(i, carry):
            c0 = i * NBUF
            for b in range(NBUF):
                c = c0 + b
                nb = (b + 2) % NBUF

                @pl.when(c >= 2)
                def _():
                    wait_write(nb)

                @pl.when(c + 2 < n_chunks)
                def _():
                    start_gather(c + 2, nb)

                wait_gather(b)
                start_write(c, b)
            return carry

        lax.fori_loop(0, n_chunks // NBUF, body, 0)
        wait_write((n_chunks - 2) % NBUF)
        wait_write((n_chunks - 1) % NBUF)

    return gather_kernel(idx_flat, table)


def kernel(x, table):
    b, l = x.shape
    _, dim = table.shape
    n_total = b * l
    n_per_c = n_total // NUM_CORES
    n_chunks = n_per_c // CHUNK
    idx_flat = x.reshape(n_total)
    out = _gather_call(idx_flat, table, n_per_c=n_per_c, n_chunks=n_chunks, dim=dim)
    return out.reshape(b, l, dim)


# speedup vs baseline: 1.2446x; 1.1547x over previous
import functools

import jax
import jax.numpy as jnp
from jax import lax
from jax.experimental import pallas as pl
from jax.experimental.pallas import tpu as pltpu
from jax.experimental.pallas import tpu_sc as plsc
from jax._src.pallas import mpmd

NUM_CORES = 2
NUM_SUBCORES = 16
NUM_WORKERS = NUM_CORES * NUM_SUBCORES

NBUF_T = 4
CHUNK_T = 24
CHUNKS_PER_TEC = 64

NBUF_S = 4
CHUNK_S = 256


@functools.partial(
    jax.jit, static_argnames=("n_tec", "n_per_w", "n_per_scs", "n_chunks_s", "dim")
)
def _gather_call(idx_flat, table, *, n_tec, n_per_w, n_per_scs, n_chunks_s, dim):
    n_total = idx_flat.shape[0]
    vmesh = plsc.VectorSubcoreMesh(core_axis_name="c", subcore_axis_name="s")
    smesh = plsc.ScalarSubcoreMesh(axis_name="c", num_cores=NUM_CORES)

    def tec_fn(
        idx_hbm,
        table_hbm,
        out_hbm,
        idx_v,
        rows_v,
        gsems,
        wsems,
        idx_s,
        spmem,
        sgsems,
        swsems,
        isems,
    ):
        wid = lax.axis_index("s") * NUM_CORES + lax.axis_index("c")
        base = wid * n_per_w
        pltpu.sync_copy(idx_hbm.at[pl.ds(base, n_per_w)], idx_v)

        bufs = tuple(rows_v.at[b] for b in range(NBUF_T))

        def start_gather(c, b):
            pltpu.async_copy(
                table_hbm.at[idx_v.at[pl.ds(c * CHUNK_T, CHUNK_T)]],
                bufs[b],
                gsems[b],
            )

        def wait_gather(b):
            pltpu.make_async_copy(
                table_hbm.at[pl.ds(0, CHUNK_T)], bufs[b], gsems[b]
            ).wait()

        def start_write(c, b):
            pltpu.async_copy(
                bufs[b], out_hbm.at[pl.ds(base + c * CHUNK_T, CHUNK_T)], wsems[b]
            )

        def wait_write(b):
            pltpu.make_async_copy(
                bufs[b], out_hbm.at[pl.ds(0, CHUNK_T)], wsems[b]
            ).wait()

        start_gather(0, 0)
        start_gather(1, 1)

        def body(i, carry):
            c0 = i * NBUF_T
            for b in range(NBUF_T):
                c = c0 + b
                nb = (b + 2) % NBUF_T

                @pl.when(c >= 2)
                def _():
                    wait_write(nb)

                @pl.when(c + 2 < CHUNKS_PER_TEC)
                def _():
                    start_gather(c + 2, nb)

                wait_gather(b)
                start_write(c, b)
            return carry

        lax.fori_loop(0, CHUNKS_PER_TEC // NBUF_T, body, 0)
        wait_write((CHUNKS_PER_TEC - 2) % NBUF_T)
        wait_write((CHUNKS_PER_TEC - 1) % NBUF_T)

    def scs_fn(
        idx_hbm,
        table_hbm,
        out_hbm,
        idx_v,
        rows_v,
        gsems,
        wsems,
        idx_s,
        spmem,
        sgsems,
        swsems,
        isems,
    ):
        cid = lax.axis_index("c")
        base = n_tec + cid * n_per_scs
        bufs = tuple(spmem.at[b] for b in range(NBUF_S))

        def prefetch_idx(c, s):
            pltpu.async_copy(
                idx_hbm.at[pl.ds(base + c * CHUNK_S, CHUNK_S)], idx_s[s], isems[s]
            )

        def wait_idx(s):
            pltpu.make_async_copy(
                idx_hbm.at[pl.ds(0, CHUNK_S)], idx_s[s], isems[s]
            ).wait()

        def start_gather(c, b):
            s = b % 2
            wait_idx(s)

            @pl.when(c + 1 < n_chunks_s)
            def _():
                prefetch_idx(c + 1, (b + 1) % 2)

            def issue(r8, carry):
                r = r8 * 8
                for u in range(8):
                    v = idx_s[s][r + u]
                    pltpu.async_copy(
                        table_hbm.at[pl.ds(v, 1)],
                        bufs[b].at[pl.ds(r + u, 1)],
                        sgsems[b],
                    )
                return carry

            lax.fori_loop(0, CHUNK_S // 8, issue, 0)

        def wait_gather(b):
            pltpu.make_async_copy(
                table_hbm.at[pl.ds(0, CHUNK_S)], bufs[b], sgsems[b]
            ).wait()

        def start_write(c, b):
            pltpu.async_copy(
                bufs[b], out_hbm.at[pl.ds(base + c * CHUNK_S, CHUNK_S)], swsems[b]
            )

        def wait_write(b):
            pltpu.make_async_copy(
                bufs[b], out_hbm.at[pl.ds(0, CHUNK_S)], swsems[b]
            ).wait()

        prefetch_idx(0, 0)
        start_gather(0, 0)
        start_gather(1, 1)

        def body(i, carry):
            c0 = i * NBUF_S
            for b in range(NBUF_S):
                c = c0 + b
                nb = (b + 2) % NBUF_S

                @pl.when(c >= 2)
                def _():
                    wait_write(nb)

                @pl.when(c + 2 < n_chunks_s)
                def _():
                    start_gather(c + 2, nb)

                wait_gather(b)
                start_write(c, b)
            return carry

        lax.fori_loop(0, n_chunks_s // NBUF_S, body, 0)
        wait_write((n_chunks_s - 2) % NBUF_S)
        wait_write((n_chunks_s - 1) % NBUF_S)

    call = mpmd.mpmd_map(
        [(smesh, scs_fn), (vmesh, tec_fn)],
        out_types=[jax.ShapeDtypeStruct((n_total, dim), jnp.float32)],
        scratch_types=[
            (pltpu.VMEM @ vmesh)((n_per_w,), jnp.int32),
            (pltpu.VMEM @ vmesh)((NBUF_T, CHUNK_T, dim), jnp.float32),
            [pltpu.SemaphoreType.DMA @ vmesh] * NBUF_T,
            [pltpu.SemaphoreType.DMA @ vmesh] * NBUF_T,
            [(pltpu.SMEM @ smesh)((CHUNK_S,), jnp.int32)] * 2,
            pltpu.VMEM_SHARED((NBUF_S, CHUNK_S, dim), jnp.float32),
            [pltpu.SemaphoreType.DMA @ smesh] * NBUF_S,
            [pltpu.SemaphoreType.DMA @ smesh] * NBUF_S,
            [pltpu.SemaphoreType.DMA @ smesh] * 2,
        ],
    )
    (out,) = call(idx_flat, table)
    return out


def kernel(x, table):
    b, l = x.shape
    _, dim = table.shape
    n_total = b * l
    n_per_w = CHUNKS_PER_TEC * CHUNK_T
    n_tec = NUM_WORKERS * n_per_w
    n_scs = n_total - n_tec
    n_per_scs = n_scs // NUM_CORES
    n_chunks_s = n_per_scs // CHUNK_S
    idx_flat = x.reshape(n_total)
    out = _gather_call(
        idx_flat,
        table,
        n_tec=n_tec,
        n_per_w=n_per_w,
        n_per_scs=n_per_scs,
        n_chunks_s=n_chunks_s,
        dim=dim,
    )
    return out.reshape(b, l, dim)


# speedup vs baseline: 1.2466x; 1.0016x over previous
import functools

import jax
import jax.numpy as jnp
from jax import lax
from jax.experimental import pallas as pl
from jax.experimental.pallas import tpu as pltpu
from jax.experimental.pallas import tpu_sc as plsc
from jax._src.pallas import mpmd

NUM_CORES = 2
NUM_SUBCORES = 16
NUM_WORKERS = NUM_CORES * NUM_SUBCORES

NBUF_T = 4
CHUNK_T = 24
CHUNKS_PER_TEC = 64

NBUF_S = 4
CHUNK_S = 256


@functools.partial(
    jax.jit, static_argnames=("n_tec", "n_per_w", "n_per_scs", "n_chunks_s", "dim")
)
def _gather_call(idx_flat, table, *, n_tec, n_per_w, n_per_scs, n_chunks_s, dim):
    n_total = idx_flat.shape[0]
    vmesh = plsc.VectorSubcoreMesh(core_axis_name="c", subcore_axis_name="s")
    smesh = plsc.ScalarSubcoreMesh(axis_name="c", num_cores=NUM_CORES)

    def tec_fn(
        idx_hbm,
        table_hbm,
        out_hbm,
        idx_s,
        spmem,
        sgsems,
        swsems,
        isems,
        gsems,
        wsems,
    ):
        pl.run_scoped(
            functools.partial(tec_body, idx_hbm, table_hbm, out_hbm, gsems, wsems),
            pltpu.VMEM((n_per_w,), jnp.int32),
            pltpu.VMEM((NBUF_T, CHUNK_T, dim), jnp.float32),
        )

    def tec_body(idx_hbm, table_hbm, out_hbm, gsems, wsems, idx_v, rows_v):
        wid = lax.axis_index("s") * NUM_CORES + lax.axis_index("c")
        base = wid * n_per_w
        pltpu.sync_copy(idx_hbm.at[pl.ds(base, n_per_w)], idx_v)

        bufs = tuple(rows_v.at[b] for b in range(NBUF_T))

        def start_gather(c, b):
            pltpu.async_copy(
                table_hbm.at[idx_v.at[pl.ds(c * CHUNK_T, CHUNK_T)]],
                bufs[b],
                gsems[b],
            )

        def wait_gather(b):
            pltpu.make_async_copy(
                table_hbm.at[pl.ds(0, CHUNK_T)], bufs[b], gsems[b]
            ).wait()

        def start_write(c, b):
            pltpu.async_copy(
                bufs[b], out_hbm.at[pl.ds(base + c * CHUNK_T, CHUNK_T)], wsems[b]
            )

        def wait_write(b):
            pltpu.make_async_copy(
                bufs[b], out_hbm.at[pl.ds(0, CHUNK_T)], wsems[b]
            ).wait()

        start_gather(0, 0)
        start_gather(1, 1)

        def body(i, carry):
            c0 = i * NBUF_T
            for b in range(NBUF_T):
                c = c0 + b
                nb = (b + 2) % NBUF_T

                @pl.when(c >= 2)
                def _():
                    wait_write(nb)

                @pl.when(c + 2 < CHUNKS_PER_TEC)
                def _():
                    start_gather(c + 2, nb)

                wait_gather(b)
                start_write(c, b)
            return carry

        lax.fori_loop(0, CHUNKS_PER_TEC // NBUF_T, body, 0)
        wait_write((CHUNKS_PER_TEC - 2) % NBUF_T)
        wait_write((CHUNKS_PER_TEC - 1) % NBUF_T)

    def scs_fn(
        idx_hbm,
        table_hbm,
        out_hbm,
        idx_s,
        spmem,
        sgsems,
        swsems,
        isems,
        gsems,
        wsems,
    ):
        cid = lax.axis_index("c")
        base = n_tec + cid * n_per_scs
        bufs = tuple(spmem.at[b] for b in range(NBUF_S))

        def prefetch_idx(c, s):
            pltpu.async_copy(
                idx_hbm.at[pl.ds(base + c * CHUNK_S, CHUNK_S)], idx_s[s], isems[s]
            )

        def wait_idx(s):
            pltpu.make_async_copy(
                idx_hbm.at[pl.ds(0, CHUNK_S)], idx_s[s], isems[s]
            ).wait()

        def start_gather(c, b):
            s = b % 2
            wait_idx(s)

            @pl.when(c + 1 < n_chunks_s)
            def _():
                prefetch_idx(c + 1, (b + 1) % 2)

            def issue(r8, carry):
                r = r8 * 8
                for u in range(8):
                    v = idx_s[s][r + u]
                    pltpu.async_copy(
                        table_hbm.at[pl.ds(v, 1)],
                        bufs[b].at[pl.ds(r + u, 1)],
                        sgsems[b],
                    )
                return carry

            lax.fori_loop(0, CHUNK_S // 8, issue, 0)

        def wait_gather(b):
            pltpu.make_async_copy(
                table_hbm.at[pl.ds(0, CHUNK_S)], bufs[b], sgsems[b]
            ).wait()

        def start_write(c, b):
            pltpu.async_copy(
                bufs[b], out_hbm.at[pl.ds(base + c * CHUNK_S, CHUNK_S)], swsems[b]
            )

        def wait_write(b):
            pltpu.make_async_copy(
                bufs[b], out_hbm.at[pl.ds(0, CHUNK_S)], swsems[b]
            ).wait()

        prefetch_idx(0, 0)
        start_gather(0, 0)
        start_gather(1, 1)

        def body(i, carry):
            c0 = i * NBUF_S
            for b in range(NBUF_S):
                c = c0 + b
                nb = (b + 2) % NBUF_S

                @pl.when(c >= 2)
                def _():
                    wait_write(nb)

                @pl.when(c + 2 < n_chunks_s)
                def _():
                    start_gather(c + 2, nb)

                wait_gather(b)
                start_write(c, b)
            return carry

        lax.fori_loop(0, n_chunks_s // NBUF_S, body, 0)
        wait_write((n_chunks_s - 2) % NBUF_S)
        wait_write((n_chunks_s - 1) % NBUF_S)

    call = mpmd.mpmd_map(
        [(smesh, scs_fn), (vmesh, tec_fn)],
        out_types=[jax.ShapeDtypeStruct((n_total, dim), jnp.float32)],
        scratch_types=[
            [(pltpu.SMEM @ smesh)((CHUNK_S,), jnp.int32)] * 2,
            pltpu.VMEM_SHARED((NBUF_S, CHUNK_S, dim), jnp.float32),
            [pltpu.SemaphoreType.DMA @ smesh] * NBUF_S,
            [pltpu.SemaphoreType.DMA @ smesh] * NBUF_S,
            [pltpu.SemaphoreType.DMA @ smesh] * 2,
            [pltpu.SemaphoreType.DMA @ vmesh] * NBUF_T,
            [pltpu.SemaphoreType.DMA @ vmesh] * NBUF_T,
        ],
    )
    (out,) = call(idx_flat, table)
    return out


def kernel(x, table):
    b, l = x.shape
    _, dim = table.shape
    n_total = b * l
    n_per_w = CHUNKS_PER_TEC * CHUNK_T
    n_tec = NUM_WORKERS * n_per_w
    n_scs = n_total - n_tec
    n_per_scs = n_scs // NUM_CORES
    n_chunks_s = n_per_scs // CHUNK_S
    idx_flat = x.reshape(n_total)
    out = _gather_call(
        idx_flat,
        table,
        n_tec=n_tec,
        n_per_w=n_per_w,
        n_per_scs=n_per_scs,
        n_chunks_s=n_chunks_s,
        dim=dim,
    )
    return out.reshape(b, l, dim)
